# baseline (speedup 1.0000x reference)
import functools

import jax
import jax.numpy as jnp
from jax.experimental import pallas as pl
from jax.experimental.pallas import tpu as pltpu
from jax.experimental.pallas import tpu_sc as plsc

F = 128
R = 20
K = 64
CUT = 1.0
N_ROWS_NBR = 8
B_EDGE = 32

_INTERPRET = False


def _silu(x):
    return x * (1.0 / (1.0 + jnp.exp(-x)))


def _nbr_body(pos_ref, post_ref, tri_ref, idx_ref, dist_ref, cnt_ref):
    rb = N_ROWS_NBR
    n = post_ref.shape[1]
    r0 = pl.program_id(0) * rb
    dx = pos_ref[:, 0:1] - post_ref[0:1, :]
    dy = pos_ref[:, 1:2] - post_ref[1:2, :]
    dz = pos_ref[:, 2:3] - post_ref[2:3, :]
    d = jnp.sqrt(dx * dx + dy * dy + dz * dz)
    col = jax.lax.broadcasted_iota(jnp.int32, (rb, n), 1)
    row = jax.lax.broadcasted_iota(jnp.int32, (rb, n), 0) + r0
    mask = (d < CUT) & (col != row)
    mf = mask.astype(jnp.float32)
    tri = tri_ref[...]
    cw = tri.shape[0]
    carry = jnp.zeros((rb, 1), jnp.float32)
    pieces = []
    for c0 in range(0, n, cw):
        chunk = mf[:, c0:c0 + cw]
        cs = jnp.dot(chunk, tri, preferred_element_type=jnp.float32) + carry
        carry = cs[:, cw - 1:cw]
        pieces.append(cs)
    slot = (jnp.concatenate(pieces, axis=1) + 0.5).astype(jnp.int32) - 1
    slot = jnp.where(mask, slot, -2)
    kio = jax.lax.broadcasted_iota(jnp.int32, (rb, n, K), 2)
    onehot = jnp.where(slot[:, :, None] == kio, 1.0, 0.0)
    idx_f = jnp.sum(onehot * col.astype(jnp.float32)[:, :, None], axis=1)
    d_f = jnp.sum(onehot * d[:, :, None], axis=1)
    cnt = jnp.sum(mf, axis=1, keepdims=True)
    idx_ref[...] = idx_f.astype(jnp.int32)
    dist_ref[...] = d_f
    cnt_ref[...] = jnp.minimum(cnt, float(K)).astype(jnp.int32)


def _build_neighbors(pos):
    n = pos.shape[0]
    grid = n // N_ROWS_NBR
    cw = 512
    tri = jnp.triu(jnp.ones((cw, cw), jnp.float32))
    return pl.pallas_call(
        _nbr_body,
        grid=(grid,),
        in_specs=[
            pl.BlockSpec((N_ROWS_NBR, 3), lambda i: (i, 0)),
            pl.BlockSpec((3, n), lambda i: (0, 0)),
            pl.BlockSpec((cw, cw), lambda i: (0, 0)),
        ],
        out_specs=[
            pl.BlockSpec((N_ROWS_NBR, K), lambda i: (i, 0)),
            pl.BlockSpec((N_ROWS_NBR, K), lambda i: (i, 0)),
            pl.BlockSpec((N_ROWS_NBR, 1), lambda i: (i, 0)),
        ],
        out_shape=[
            jax.ShapeDtypeStruct((n, K), jnp.int32),
            jax.ShapeDtypeStruct((n, K), jnp.float32),
            jax.ShapeDtypeStruct((n, 1), jnp.int32),
        ],
        interpret=_INTERPRET,
    )(pos, pos.T, tri)


def _gather_rows(table, idx):
    num = idx.shape[0]
    width = table.shape[1]
    window = 128
    idx2 = idx.reshape(1, num)
    mesh = plsc.VectorSubcoreMesh(core_axis_name="core", subcore_axis_name="subcore")

    @pl.kernel(out_type=jax.ShapeDtypeStruct((num, width), table.dtype), mesh=mesh)
    def kern(x_hbm, i_hbm, o_hbm):
        def body(i_vmem, o_vmem):
            pltpu.sync_copy(x_hbm.at[i_vmem.at[0]], o_vmem)

        pltpu.emit_pipeline(
            body,
            grid=(num // window,),
            in_specs=[pl.BlockSpec((1, window), index_map=lambda i: (0, i))],
            out_specs=[pl.BlockSpec((window, width), index_map=lambda i: (i, 0))],
            core_axis_name=("core", "subcore"),
            dimension_semantics=(pltpu.PARALLEL,),
        )(i_hbm, o_hbm)

    return kern(table, idx2)


def _pre_body(s_ref, w1ss, b1s, w1vs, b1v, as_ref, av_ref):
    s = s_ref[...]
    as_ref[...] = jnp.dot(s, w1ss[...], preferred_element_type=jnp.float32) + b1s[...]
    av_ref[...] = jnp.dot(s, w1vs[...], preferred_element_type=jnp.float32) + b1v[...]


def _build_table(s, w1ss, b1s, w1vs, b1v):
    n = s.shape[0]
    blk = 512
    return pl.pallas_call(
        _pre_body,
        grid=(n // blk,),
        in_specs=[
            pl.BlockSpec((blk, F), lambda i: (i, 0)),
            pl.BlockSpec((F, F), lambda i: (0, 0)),
            pl.BlockSpec((1, F), lambda i: (0, 0)),
            pl.BlockSpec((F, F), lambda i: (0, 0)),
            pl.BlockSpec((1, F), lambda i: (0, 0)),
        ],
        out_specs=[
            pl.BlockSpec((blk, F), lambda i: (i, 0)),
            pl.BlockSpec((blk, F), lambda i: (i, 0)),
        ],
        out_shape=[
            jax.ShapeDtypeStruct((n, F), jnp.float32),
            jax.ShapeDtypeStruct((n, F), jnp.float32),
        ],
        interpret=_INTERPRET,
    )(s, w1ss, b1s, w1vs, b1v)


def _edge_body(gas_ref, gav_ref, gv0_ref, gv1_ref, gv2_ref, dist_ref, cnt_ref, vj_ref,
               w1sr, w2s, b2s, w1vr, w2v, b2v, ds_ref, dv_ref):
    b = B_EDGE
    e = b * K
    d = dist_ref[...]
    sig = CUT / (R - 1.0)
    mu = jax.lax.broadcasted_iota(jnp.int32, (1, R), 1).astype(jnp.float32) * sig
    rbf = jnp.exp(-((d - mu) ** 2) / (2.0 * sig * sig)) * (d < CUT).astype(jnp.float32)
    es = gas_ref[...]
    ev = gav_ref[...]
    gv_refs = (gv0_ref, gv1_ref, gv2_ref)
    ps = jnp.dot(rbf, w1sr[...], preferred_element_type=jnp.float32)
    pv = jnp.dot(rbf, w1vr[...], preferred_element_type=jnp.float32)
    hs = _silu(es + ps)
    ds_e = jnp.dot(hs, w2s[...], preferred_element_type=jnp.float32) + b2s[...]
    hv = _silu(ev + pv)
    g = jnp.dot(hv, w2v[...], preferred_element_type=jnp.float32) + b2v[...]

    kk = jax.lax.broadcasted_iota(jnp.int32, (e, 1), 0) % K
    valid = (kk < cnt_ref[...]).astype(jnp.float32)
    ds_e = ds_e * valid
    ds_ref[...] = jnp.sum(ds_e.reshape(b, K, F), axis=1)

    vj = vj_ref[...]
    diffs = []
    n2 = jnp.zeros((e, F), jnp.float32)
    for dd in range(3):
        vi_d = gv_refs[dd][...]
        vj_d = vj[:, dd * F:(dd + 1) * F]
        vj_rep = jnp.broadcast_to(vj_d[:, None, :], (b, K, F)).reshape(e, F)
        df = vi_d - vj_rep
        diffs.append(df)
        n2 = n2 + df * df
    inv = 1.0 / (jnp.sqrt(n2) + 1e-8)
    gm = g * valid
    for dd in range(3):
        dv_d = gm * diffs[dd] * inv
        dv_ref[:, dd * F:(dd + 1) * F] = jnp.sum(dv_d.reshape(b, K, F), axis=1)


def _edge_compute(ga_s, ga_v, gv0, gv1, gv2, nbr_dist, nbr_cnt, v_cat,
                  w1sr, w2s, b2s, w1vr, w2v, b2v):
    n = v_cat.shape[0]
    grid = n // B_EDGE
    espec = pl.BlockSpec((B_EDGE * K, F), lambda i: (i, 0))
    return pl.pallas_call(
        _edge_body,
        grid=(grid,),
        in_specs=[
            espec, espec, espec, espec, espec,
            pl.BlockSpec((B_EDGE * K, 1), lambda i: (i, 0)),
            pl.BlockSpec((B_EDGE * K, 1), lambda i: (i, 0)),
            pl.BlockSpec((B_EDGE, 3 * F), lambda i: (i, 0)),
            pl.BlockSpec((R, F), lambda i: (0, 0)),
            pl.BlockSpec((F, F), lambda i: (0, 0)),
            pl.BlockSpec((1, F), lambda i: (0, 0)),
            pl.BlockSpec((R, F), lambda i: (0, 0)),
            pl.BlockSpec((F, F), lambda i: (0, 0)),
            pl.BlockSpec((1, F), lambda i: (0, 0)),
        ],
        out_specs=[
            pl.BlockSpec((B_EDGE, F), lambda i: (i, 0)),
            pl.BlockSpec((B_EDGE, 3 * F), lambda i: (i, 0)),
        ],
        out_shape=[
            jax.ShapeDtypeStruct((n, F), jnp.float32),
            jax.ShapeDtypeStruct((n, 3 * F), jnp.float32),
        ],
        interpret=_INTERPRET,
    )(ga_s, ga_v, gv0, gv1, gv2, nbr_dist.reshape(-1, 1), jnp.repeat(nbr_cnt, K, axis=0),
      v_cat, w1sr, w2s, b2s, w1vr, w2v, b2v)


def _upd_body(s_ref, v_ref, ds_ref, dv_ref, ua, ub, bu1, u2, bu2,
              va, vb, bv1, v2w, bv2, so_ref, vo_ref):
    s = s_ref[...]
    hs = _silu(jnp.dot(s, ua[...], preferred_element_type=jnp.float32)
               + jnp.dot(ds_ref[...], ub[...], preferred_element_type=jnp.float32) + bu1[...])
    so_ref[...] = s + jnp.dot(hs, u2[...], preferred_element_type=jnp.float32) + bu2[...]
    for dd in range(3):
        vd = v_ref[:, dd * F:(dd + 1) * F]
        dvd = dv_ref[:, dd * F:(dd + 1) * F]
        hv = _silu(jnp.dot(vd, va[...], preferred_element_type=jnp.float32)
                   + jnp.dot(dvd, vb[...], preferred_element_type=jnp.float32) + bv1[...])
        vo_ref[:, dd * F:(dd + 1) * F] = vd + jnp.dot(hv, v2w[...], preferred_element_type=jnp.float32) + bv2[...]


def _update(s, v_cat, ds_agg, dv_agg, ua, ub, bu1, u2, bu2, va, vb, bv1, v2w, bv2):
    n = s.shape[0]
    blk = 512
    wspec = pl.BlockSpec((F, F), lambda i: (0, 0))
    bspec = pl.BlockSpec((1, F), lambda i: (0, 0))
    return pl.pallas_call(
        _upd_body,
        grid=(n // blk,),
        in_specs=[
            pl.BlockSpec((blk, F), lambda i: (i, 0)),
            pl.BlockSpec((blk, 3 * F), lambda i: (i, 0)),
            pl.BlockSpec((blk, F), lambda i: (i, 0)),
            pl.BlockSpec((blk, 3 * F), lambda i: (i, 0)),
            wspec, wspec, bspec, wspec, bspec,
            wspec, wspec, bspec, wspec, bspec,
        ],
        out_specs=[
            pl.BlockSpec((blk, F), lambda i: (i, 0)),
            pl.BlockSpec((blk, 3 * F), lambda i: (i, 0)),
        ],
        out_shape=[
            jax.ShapeDtypeStruct((n, F), jnp.float32),
            jax.ShapeDtypeStruct((n, 3 * F), jnp.float32),
        ],
        interpret=_INTERPRET,
    )(s, v_cat, ds_agg, dv_agg, ua, ub, bu1, u2, bu2, va, vb, bv1, v2w, bv2)


def _init_body(s_ref, w, b, v_ref):
    v_ref[...] = jnp.dot(s_ref[...], w[...], preferred_element_type=jnp.float32) + b[...]


def _init_vector(s, w, b):
    n = s.shape[0]
    blk = 512
    return pl.pallas_call(
        _init_body,
        grid=(n // blk,),
        in_specs=[
            pl.BlockSpec((blk, F), lambda i: (i, 0)),
            pl.BlockSpec((F, 3 * F), lambda i: (0, 0)),
            pl.BlockSpec((1, 3 * F), lambda i: (0, 0)),
        ],
        out_specs=pl.BlockSpec((blk, 3 * F), lambda i: (i, 0)),
        out_shape=jax.ShapeDtypeStruct((n, 3 * F), jnp.float32),
        interpret=_INTERPRET,
    )(s, w, b)


def _out_body(s_ref, w1, b1, w2, b2, w3, b3, o_ref):
    x = _silu(jnp.dot(s_ref[...], w1[...], preferred_element_type=jnp.float32) + b1[...])
    x = _silu(jnp.dot(x, w2[...], preferred_element_type=jnp.float32) + b2[...])
    o_ref[...] = jnp.dot(x, w3[...], preferred_element_type=jnp.float32) + b3[...]


def _out_head(s, w1, b1, w2, b2, w3, b3):
    n = s.shape[0]
    blk = 512
    wspec = pl.BlockSpec((F, F), lambda i: (0, 0))
    bspec = pl.BlockSpec((1, F), lambda i: (0, 0))
    return pl.pallas_call(
        _out_body,
        grid=(n // blk,),
        in_specs=[
            pl.BlockSpec((blk, F), lambda i: (i, 0)),
            wspec, bspec, wspec, bspec,
            pl.BlockSpec((F, 1), lambda i: (0, 0)),
            pl.BlockSpec((1, 1), lambda i: (0, 0)),
        ],
        out_specs=pl.BlockSpec((blk, 1), lambda i: (i, 0)),
        out_shape=jax.ShapeDtypeStruct((n, 1), jnp.float32),
        interpret=_INTERPRET,
    )(s, w1, b1, w2, b2, w3, b3)


def kernel(atoms, atom_positions, graph_indexes, params):
    n = atoms.shape[0]
    pos = atom_positions.astype(jnp.float32)

    nbr_idx, nbr_dist, nbr_cnt = _build_neighbors(pos)
    flat_idx = nbr_idx.reshape(-1)

    s = _gather_rows(params['embedding'], atoms.astype(jnp.int32))
    wiv, biv = params['init_vector']
    v_cat = _init_vector(s, wiv, biv.reshape(1, 3 * F))

    for bp in params['blocks']:
        (w1s, b1s_), (w2s, b2s_) = bp['msg_scalar']
        (w1v, b1v_), (w2v, b2v_) = bp['msg_vector']
        (wu1, bu1_), (wu2, bu2_) = bp['upd_scalar']
        (wv1, bv1_), (wv2, bv2_) = bp['upd_vector']

        tab_as, tab_av = _build_table(s,
                                      w1s[:F], b1s_.reshape(1, F),
                                      w1v[:F], b1v_.reshape(1, F))
        ga_s = _gather_rows(tab_as, flat_idx)
        ga_v = _gather_rows(tab_av, flat_idx)
        gv0 = _gather_rows(v_cat[:, 0:F], flat_idx)
        gv1 = _gather_rows(v_cat[:, F:2 * F], flat_idx)
        gv2 = _gather_rows(v_cat[:, 2 * F:], flat_idx)
        ds_agg, dv_agg = _edge_compute(ga_s, ga_v, gv0, gv1, gv2, nbr_dist, nbr_cnt, v_cat,
                                       w1s[F:], w2s, b2s_.reshape(1, F),
                                       w1v[F:], w2v, b2v_.reshape(1, F))
        s, v_cat = _update(s, v_cat, ds_agg, dv_agg,
                           wu1[:F], wu1[F:], bu1_.reshape(1, F), wu2, bu2_.reshape(1, F),
                           wv1[:F], wv1[F:], bv1_.reshape(1, F), wv2, bv2_.reshape(1, F))

    (o1w, o1b), (o2w, o2b), (o3w, o3b) = params['output']
    return _out_head(s, o1w, o1b.reshape(1, F), o2w, o2b.reshape(1, F),
                     o3w.reshape(F, 1), o3b.reshape(1, 1))


# speedup vs baseline: 9.5273x; 9.5273x over previous
import functools

import jax
import jax.numpy as jnp
from jax.experimental import pallas as pl
from jax.experimental.pallas import tpu as pltpu
from jax.experimental.pallas import tpu_sc as plsc

F = 128
R = 20
K = 64
CUT = 1.0
N_ROWS_NBR = 8
B_EDGE = 32

_INTERPRET = False


def _silu(x):
    return x * (1.0 / (1.0 + jnp.exp(-x)))


def _nbr_body(pos_ref, post_ref, tri_ref, idx_ref, dist_ref, cnt_ref):
    rb = N_ROWS_NBR
    n = post_ref.shape[1]
    r0 = pl.program_id(0) * rb
    dx = pos_ref[:, 0:1] - post_ref[0:1, :]
    dy = pos_ref[:, 1:2] - post_ref[1:2, :]
    dz = pos_ref[:, 2:3] - post_ref[2:3, :]
    d = jnp.sqrt(dx * dx + dy * dy + dz * dz)
    col = jax.lax.broadcasted_iota(jnp.int32, (rb, n), 1)
    row = jax.lax.broadcasted_iota(jnp.int32, (rb, n), 0) + r0
    mask = (d < CUT) & (col != row)
    mf = mask.astype(jnp.float32)
    tri = tri_ref[...]
    cw = tri.shape[0]
    carry = jnp.zeros((rb, 1), jnp.float32)
    pieces = []
    for c0 in range(0, n, cw):
        chunk = mf[:, c0:c0 + cw]
        cs = jnp.dot(chunk, tri, preferred_element_type=jnp.float32) + carry
        carry = cs[:, cw - 1:cw]
        pieces.append(cs)
    slot = (jnp.concatenate(pieces, axis=1) + 0.5).astype(jnp.int32) - 1
    slot = jnp.where(mask, slot, -2)
    kio = jax.lax.broadcasted_iota(jnp.int32, (rb, n, K), 2)
    onehot = jnp.where(slot[:, :, None] == kio, 1.0, 0.0)
    idx_f = jnp.sum(onehot * col.astype(jnp.float32)[:, :, None], axis=1)
    d_f = jnp.sum(onehot * d[:, :, None], axis=1)
    cnt = jnp.sum(mf, axis=1, keepdims=True)
    fill = jnp.sum(onehot, axis=1)
    own = (jax.lax.broadcasted_iota(jnp.int32, (rb, K), 0) + r0).astype(jnp.float32)
    idx_f = idx_f + (1.0 - fill) * own
    idx_ref[...] = idx_f.astype(jnp.int32)
    dist_ref[...] = d_f
    cnt_ref[...] = jnp.minimum(cnt, float(K)).astype(jnp.int32)


def _build_neighbors(pos):
    n = pos.shape[0]
    grid = n // N_ROWS_NBR
    cw = 512
    tri = jnp.triu(jnp.ones((cw, cw), jnp.float32))
    return pl.pallas_call(
        _nbr_body,
        grid=(grid,),
        in_specs=[
            pl.BlockSpec((N_ROWS_NBR, 3), lambda i: (i, 0)),
            pl.BlockSpec((3, n), lambda i: (0, 0)),
            pl.BlockSpec((cw, cw), lambda i: (0, 0)),
        ],
        out_specs=[
            pl.BlockSpec((N_ROWS_NBR, K), lambda i: (i, 0)),
            pl.BlockSpec((N_ROWS_NBR, K), lambda i: (i, 0)),
            pl.BlockSpec((N_ROWS_NBR, 1), lambda i: (i, 0)),
        ],
        out_shape=[
            jax.ShapeDtypeStruct((n, K), jnp.int32),
            jax.ShapeDtypeStruct((n, K), jnp.float32),
            jax.ShapeDtypeStruct((n, 1), jnp.int32),
        ],
        interpret=_INTERPRET,
    )(pos, pos.T, tri)


def _gather_rows(table, idx):
    num = idx.shape[0]
    width = table.shape[1]
    window = 128
    idx2 = idx.reshape(1, num)
    mesh = plsc.VectorSubcoreMesh(core_axis_name="core", subcore_axis_name="subcore")

    @pl.kernel(out_type=jax.ShapeDtypeStruct((num, width), table.dtype), mesh=mesh)
    def kern(x_hbm, i_hbm, o_hbm):
        def body(i_vmem, o_vmem):
            pltpu.sync_copy(x_hbm.at[i_vmem.at[0]], o_vmem)

        pltpu.emit_pipeline(
            body,
            grid=(num // window,),
            in_specs=[pl.BlockSpec((1, window), index_map=lambda i: (0, i))],
            out_specs=[pl.BlockSpec((window, width), index_map=lambda i: (i, 0))],
            core_axis_name=("core", "subcore"),
            dimension_semantics=(pltpu.PARALLEL,),
        )(i_hbm, o_hbm)

    return kern(table, idx2)


def _pre_body(s_ref, w1ss, b1s, w1vs, b1v, as_ref, av_ref):
    s = s_ref[...]
    as_ref[...] = jnp.dot(s, w1ss[...], preferred_element_type=jnp.float32) + b1s[...]
    av_ref[...] = jnp.dot(s, w1vs[...], preferred_element_type=jnp.float32) + b1v[...]


def _build_table(s, w1ss, b1s, w1vs, b1v):
    n = s.shape[0]
    blk = 512
    return pl.pallas_call(
        _pre_body,
        grid=(n // blk,),
        in_specs=[
            pl.BlockSpec((blk, F), lambda i: (i, 0)),
            pl.BlockSpec((F, F), lambda i: (0, 0)),
            pl.BlockSpec((1, F), lambda i: (0, 0)),
            pl.BlockSpec((F, F), lambda i: (0, 0)),
            pl.BlockSpec((1, F), lambda i: (0, 0)),
        ],
        out_specs=[
            pl.BlockSpec((blk, F), lambda i: (i, 0)),
            pl.BlockSpec((blk, F), lambda i: (i, 0)),
        ],
        out_shape=[
            jax.ShapeDtypeStruct((n, F), jnp.float32),
            jax.ShapeDtypeStruct((n, F), jnp.float32),
        ],
        interpret=_INTERPRET,
    )(s, w1ss, b1s, w1vs, b1v)


def _edge_body(gas_ref, gav_ref, gv0_ref, gv1_ref, gv2_ref, dist_ref, cnt_ref, vj_ref,
               w1sr, w2s, b2s, w1vr, w2v, b2v, ds_ref, dv_ref):
    b = B_EDGE
    e = b * K
    d = dist_ref[...]
    sig = CUT / (R - 1.0)
    mu = jax.lax.broadcasted_iota(jnp.int32, (1, R), 1).astype(jnp.float32) * sig
    rbf = jnp.exp(-((d - mu) ** 2) / (2.0 * sig * sig)) * (d < CUT).astype(jnp.float32)
    es = gas_ref[...]
    ev = gav_ref[...]
    gv_refs = (gv0_ref, gv1_ref, gv2_ref)
    ps = jnp.dot(rbf, w1sr[...], preferred_element_type=jnp.float32)
    pv = jnp.dot(rbf, w1vr[...], preferred_element_type=jnp.float32)
    hs = _silu(es + ps)
    ds_e = jnp.dot(hs, w2s[...], preferred_element_type=jnp.float32) + b2s[...]
    hv = _silu(ev + pv)
    g = jnp.dot(hv, w2v[...], preferred_element_type=jnp.float32) + b2v[...]

    kk = jax.lax.broadcasted_iota(jnp.int32, (e, 1), 0) % K
    valid = (kk < cnt_ref[...]).astype(jnp.float32)
    ds_e = ds_e * valid
    ds_ref[...] = jnp.sum(ds_e.reshape(b, K, F), axis=1)

    vj = vj_ref[...]
    diffs = []
    n2 = jnp.zeros((e, F), jnp.float32)
    for dd in range(3):
        vi_d = gv_refs[dd][...]
        vj_d = vj[:, dd * F:(dd + 1) * F]
        vj_rep = jnp.broadcast_to(vj_d[:, None, :], (b, K, F)).reshape(e, F)
        df = vi_d - vj_rep
        diffs.append(df)
        n2 = n2 + df * df
    inv = 1.0 / (jnp.sqrt(n2) + 1e-8)
    gm = g * valid
    for dd in range(3):
        dv_d = gm * diffs[dd] * inv
        dv_ref[:, dd * F:(dd + 1) * F] = jnp.sum(dv_d.reshape(b, K, F), axis=1)


def _edge_compute(ga_s, ga_v, gv0, gv1, gv2, nbr_dist, nbr_cnt, v_cat,
                  w1sr, w2s, b2s, w1vr, w2v, b2v):
    n = v_cat.shape[0]
    grid = n // B_EDGE
    espec = pl.BlockSpec((B_EDGE * K, F), lambda i: (i, 0))
    return pl.pallas_call(
        _edge_body,
        grid=(grid,),
        in_specs=[
            espec, espec, espec, espec, espec,
            pl.BlockSpec((B_EDGE * K, 1), lambda i: (i, 0)),
            pl.BlockSpec((B_EDGE * K, 1), lambda i: (i, 0)),
            pl.BlockSpec((B_EDGE, 3 * F), lambda i: (i, 0)),
            pl.BlockSpec((R, F), lambda i: (0, 0)),
            pl.BlockSpec((F, F), lambda i: (0, 0)),
            pl.BlockSpec((1, F), lambda i: (0, 0)),
            pl.BlockSpec((R, F), lambda i: (0, 0)),
            pl.BlockSpec((F, F), lambda i: (0, 0)),
            pl.BlockSpec((1, F), lambda i: (0, 0)),
        ],
        out_specs=[
            pl.BlockSpec((B_EDGE, F), lambda i: (i, 0)),
            pl.BlockSpec((B_EDGE, 3 * F), lambda i: (i, 0)),
        ],
        out_shape=[
            jax.ShapeDtypeStruct((n, F), jnp.float32),
            jax.ShapeDtypeStruct((n, 3 * F), jnp.float32),
        ],
        interpret=_INTERPRET,
    )(ga_s, ga_v, gv0, gv1, gv2, nbr_dist.reshape(-1, 1), jnp.repeat(nbr_cnt, K, axis=0),
      v_cat, w1sr, w2s, b2s, w1vr, w2v, b2v)


def _upd_body(s_ref, v_ref, ds_ref, dv_ref, ua, ub, bu1, u2, bu2,
              va, vb, bv1, v2w, bv2, so_ref, vo_ref):
    s = s_ref[...]
    hs = _silu(jnp.dot(s, ua[...], preferred_element_type=jnp.float32)
               + jnp.dot(ds_ref[...], ub[...], preferred_element_type=jnp.float32) + bu1[...])
    so_ref[...] = s + jnp.dot(hs, u2[...], preferred_element_type=jnp.float32) + bu2[...]
    for dd in range(3):
        vd = v_ref[:, dd * F:(dd + 1) * F]
        dvd = dv_ref[:, dd * F:(dd + 1) * F]
        hv = _silu(jnp.dot(vd, va[...], preferred_element_type=jnp.float32)
                   + jnp.dot(dvd, vb[...], preferred_element_type=jnp.float32) + bv1[...])
        vo_ref[:, dd * F:(dd + 1) * F] = vd + jnp.dot(hv, v2w[...], preferred_element_type=jnp.float32) + bv2[...]


def _update(s, v_cat, ds_agg, dv_agg, ua, ub, bu1, u2, bu2, va, vb, bv1, v2w, bv2):
    n = s.shape[0]
    blk = 512
    wspec = pl.BlockSpec((F, F), lambda i: (0, 0))
    bspec = pl.BlockSpec((1, F), lambda i: (0, 0))
    return pl.pallas_call(
        _upd_body,
        grid=(n // blk,),
        in_specs=[
            pl.BlockSpec((blk, F), lambda i: (i, 0)),
            pl.BlockSpec((blk, 3 * F), lambda i: (i, 0)),
            pl.BlockSpec((blk, F), lambda i: (i, 0)),
            pl.BlockSpec((blk, 3 * F), lambda i: (i, 0)),
            wspec, wspec, bspec, wspec, bspec,
            wspec, wspec, bspec, wspec, bspec,
        ],
        out_specs=[
            pl.BlockSpec((blk, F), lambda i: (i, 0)),
            pl.BlockSpec((blk, 3 * F), lambda i: (i, 0)),
        ],
        out_shape=[
            jax.ShapeDtypeStruct((n, F), jnp.float32),
            jax.ShapeDtypeStruct((n, 3 * F), jnp.float32),
        ],
        interpret=_INTERPRET,
    )(s, v_cat, ds_agg, dv_agg, ua, ub, bu1, u2, bu2, va, vb, bv1, v2w, bv2)


def _init_body(s_ref, w, b, v_ref):
    v_ref[...] = jnp.dot(s_ref[...], w[...], preferred_element_type=jnp.float32) + b[...]


def _init_vector(s, w, b):
    n = s.shape[0]
    blk = 512
    return pl.pallas_call(
        _init_body,
        grid=(n // blk,),
        in_specs=[
            pl.BlockSpec((blk, F), lambda i: (i, 0)),
            pl.BlockSpec((F, 3 * F), lambda i: (0, 0)),
            pl.BlockSpec((1, 3 * F), lambda i: (0, 0)),
        ],
        out_specs=pl.BlockSpec((blk, 3 * F), lambda i: (i, 0)),
        out_shape=jax.ShapeDtypeStruct((n, 3 * F), jnp.float32),
        interpret=_INTERPRET,
    )(s, w, b)


def _out_body(s_ref, w1, b1, w2, b2, w3, b3, o_ref):
    x = _silu(jnp.dot(s_ref[...], w1[...], preferred_element_type=jnp.float32) + b1[...])
    x = _silu(jnp.dot(x, w2[...], preferred_element_type=jnp.float32) + b2[...])
    o_ref[...] = jnp.dot(x, w3[...], preferred_element_type=jnp.float32) + b3[...]


def _out_head(s, w1, b1, w2, b2, w3, b3):
    n = s.shape[0]
    blk = 512
    wspec = pl.BlockSpec((F, F), lambda i: (0, 0))
    bspec = pl.BlockSpec((1, F), lambda i: (0, 0))
    return pl.pallas_call(
        _out_body,
        grid=(n // blk,),
        in_specs=[
            pl.BlockSpec((blk, F), lambda i: (i, 0)),
            wspec, bspec, wspec, bspec,
            pl.BlockSpec((F, 1), lambda i: (0, 0)),
            pl.BlockSpec((1, 1), lambda i: (0, 0)),
        ],
        out_specs=pl.BlockSpec((blk, 1), lambda i: (i, 0)),
        out_shape=jax.ShapeDtypeStruct((n, 1), jnp.float32),
        interpret=_INTERPRET,
    )(s, w1, b1, w2, b2, w3, b3)


def kernel(atoms, atom_positions, graph_indexes, params):
    n = atoms.shape[0]
    pos = atom_positions.astype(jnp.float32)

    nbr_idx, nbr_dist, nbr_cnt = _build_neighbors(pos)
    flat_idx = nbr_idx.reshape(-1)

    s = _gather_rows(params['embedding'], atoms.astype(jnp.int32))
    wiv, biv = params['init_vector']
    v_cat = _init_vector(s, wiv, biv.reshape(1, 3 * F))

    for bp in params['blocks']:
        (w1s, b1s_), (w2s, b2s_) = bp['msg_scalar']
        (w1v, b1v_), (w2v, b2v_) = bp['msg_vector']
        (wu1, bu1_), (wu2, bu2_) = bp['upd_scalar']
        (wv1, bv1_), (wv2, bv2_) = bp['upd_vector']

        tab_as, tab_av = _build_table(s,
                                      w1s[:F], b1s_.reshape(1, F),
                                      w1v[:F], b1v_.reshape(1, F))
        ga_s = _gather_rows(tab_as, flat_idx)
        ga_v = _gather_rows(tab_av, flat_idx)
        gv0 = _gather_rows(v_cat[:, 0:F], flat_idx)
        gv1 = _gather_rows(v_cat[:, F:2 * F], flat_idx)
        gv2 = _gather_rows(v_cat[:, 2 * F:], flat_idx)
        ds_agg, dv_agg = _edge_compute(ga_s, ga_v, gv0, gv1, gv2, nbr_dist, nbr_cnt, v_cat,
                                       w1s[F:], w2s, b2s_.reshape(1, F),
                                       w1v[F:], w2v, b2v_.reshape(1, F))
        s, v_cat = _update(s, v_cat, ds_agg, dv_agg,
                           wu1[:F], wu1[F:], bu1_.reshape(1, F), wu2, bu2_.reshape(1, F),
                           wv1[:F], wv1[F:], bv1_.reshape(1, F), wv2, bv2_.reshape(1, F))

    (o1w, o1b), (o2w, o2b), (o3w, o3b) = params['output']
    return _out_head(s, o1w, o1b.reshape(1, F), o2w, o2b.reshape(1, F),
                     o3w.reshape(F, 1), o3b.reshape(1, 1))
